# baseline (speedup 1.0000x reference)
import functools

import jax
import jax.numpy as jnp
from jax import lax
from jax.experimental import pallas as pl
from jax.experimental.pallas import tpu as pltpu
from jax.experimental.pallas import tpu_sc as plsc

MIN_NORM = 1e-15
EPS = 1e-5
N_USERS = 50000
N_ITEMS = 30000
N_ENTITIES = 50000
N_EDGES = 800000
NNZ = 800000
N_REL = 16
D = 64

NC, NS = 2, 16
NW = NC * NS
GCH = 1024
GCR = GCH // 128
GCHUNKS = 25
EP = NW * GCHUNKS * GCH
PER_SC = EP // NS

ENT_HALF = 25000
ENT_ROWS = 25088
ENT_STRIPE = ENT_ROWS // NS
ENT_TRASH = 25040
ITEM_HALF = 15000
ITEM_ROWS = 15104
ITEM_STRIPE = ITEM_ROWS // NS
ITEM_TRASH = 15040
USR_HALF = 25000
USR_ROWS = 25088
USR_STRIPE = 1568
USR_TRASH = 25040

CNT_BINS = 50176

_MESH = plsc.VectorSubcoreMesh(core_axis_name="c", subcore_axis_name="s")
_SC_PARAMS = pltpu.CompilerParams(use_tc_tiling_on_sc=False,
                                  needs_layout_passes=False)


def _wid():
    return lax.axis_index("s") * NC + lax.axis_index("c")


GW = 512
GWR = GW // 128


@functools.partial(
    pl.kernel,
    out_type=[jax.ShapeDtypeStruct((EP, D), jnp.float32),
              jax.ShapeDtypeStruct((EP, D), jnp.float32),
              jax.ShapeDtypeStruct((NW, CNT_BINS), jnp.float32)],
    mesh=_MESH,
    compiler_params=_SC_PARAMS,
    scratch_types=[pltpu.VMEM((GWR, 128), jnp.int32),
                   pltpu.VMEM((GWR, 128), jnp.int32),
                   pltpu.VMEM((2 * GWR, 128), jnp.int32),
                   pltpu.VMEM((CNT_BINS,), jnp.float32),
                   pltpu.VMEM((GW, D), jnp.float32),
                   pltpu.VMEM((GW, D), jnp.float32),
                   pltpu.SemaphoreType.DMA,
                   pltpu.SemaphoreType.DMA],
)
def _gather_ht(tab, hidx, tidx, hidx_s, zero_cnt, out_h, out_t, out_cnt,
               ib0, ib1, ibs, hist, b0, b1, semg, semw):
    w = _wid()
    base_r = w * (EP // NW // 128)
    pltpu.sync_copy(zero_cnt, hist)
    ones = jnp.ones((16,), jnp.float32)

    for idx_ref, out_ref, do_hist in ((hidx, out_h, True),
                                      (tidx, out_t, False)):

        @pl.loop(0, 25)
        def _i(i):
            r0 = base_r + i * 2 * GWR
            pltpu.sync_copy(idx_ref.at[pl.ds(r0, GWR)], ib0)
            pltpu.sync_copy(idx_ref.at[pl.ds(r0 + GWR, GWR)], ib1)
            if do_hist:
                pltpu.sync_copy(hidx_s.at[pl.ds(r0, 2 * GWR)], ibs)

            @pl.when(i > 0)
            def _drain_writes():
                pltpu.make_async_copy(b0, out_ref.at[pl.ds(0, GW)],
                                      semw).wait()
                pltpu.make_async_copy(b1, out_ref.at[pl.ds(0, GW)],
                                      semw).wait()

            descs = []
            for ib, buf in ((ib0, b0), (ib1, b1)):
                for j in range(GWR):
                    descs.append(
                        pltpu.async_copy(tab.at[ib.at[j]],
                                         buf.at[pl.ds(j * 128, 128)], semg))
            if do_hist:
                for j in range(2 * GWR):

                    @pl.loop(0, 8, unroll=2)
                    def _g(g):
                        v = ibs[j, pl.ds(g * 16, 16)]
                        plsc.addupdate_scatter(hist, [v], ones)

            for d in descs:
                d.wait()
            pltpu.async_copy(b0, out_ref.at[pl.ds(r0 * 128, GW)], semw)
            pltpu.async_copy(b1, out_ref.at[pl.ds((r0 + GWR) * 128, GW)],
                             semw)

        pltpu.make_async_copy(b0, out_ref.at[pl.ds(0, GW)], semw).wait()
        pltpu.make_async_copy(b1, out_ref.at[pl.ds(0, GW)], semw).wait()

    pltpu.sync_copy(hist, out_cnt.at[w])


ECH = 256
ECR = ECH // 128


@functools.partial(
    pl.kernel,
    out_type=jax.ShapeDtypeStruct((NC, ENT_ROWS, D), jnp.float32),
    mesh=_MESH,
    compiler_params=_SC_PARAMS,
    scratch_types=[pltpu.VMEM((ECH, D), jnp.float32),
                   pltpu.VMEM((ECR, 128), jnp.int32),
                   pltpu.VMEM((ECR, 128), jnp.int32),
                   pltpu.VMEM_SHARED((ENT_ROWS, D), jnp.float32),
                   pltpu.SemaphoreType.DMA],
)
def _ent_scatter(res, hidx, zero64, sums_out, rows, rawi, locv, sums_acc,
                 semw):
    c = lax.axis_index("c")
    s = lax.axis_index("s")
    s0 = s * ENT_STRIPE
    pltpu.sync_copy(zero64, sums_acc.at[pl.ds(s0, ENT_STRIPE)])
    plsc.subcore_barrier()

    half0 = c * ENT_HALF

    @pl.loop(0, PER_SC // ECH)
    def _chunk(k):
        base = s * PER_SC + k * ECH
        r0 = base // 128
        pltpu.sync_copy(res.at[pl.ds(base, ECH)], rows)
        pltpu.sync_copy(hidx.at[pl.ds(r0, ECR)], rawi)
        for j in range(ECR):
            for g in range(8):
                v = rawi[j, pl.ds(g * 16, 16)]
                lv = v - half0
                ok = (lv >= 0) & (lv < ENT_HALF)
                locv[j, pl.ds(g * 16, 16)] = jnp.where(ok, lv, ENT_TRASH)
        sd = [
            pltpu.async_copy(rows.at[pl.ds(j * 128, 128)],
                             sums_acc.at[locv.at[j]], semw, add=True)
            for j in range(ECR)
        ]
        for d in sd:
            d.wait()

    plsc.subcore_barrier()
    pltpu.sync_copy(sums_acc.at[pl.ds(s0, ENT_STRIPE)],
                    sums_out.at[c, pl.ds(s0, ENT_STRIPE)])


ICH = 512
ICR = ICH // 128


@functools.partial(
    pl.kernel,
    out_type=jax.ShapeDtypeStruct((NC, ITEM_ROWS, D), jnp.float32),
    mesh=_MESH,
    compiler_params=_SC_PARAMS,
    scratch_types=[pltpu.VMEM((ICH, D), jnp.float32),
                   pltpu.VMEM((ICH, D), jnp.float32),
                   pltpu.VMEM((ICR, 128), jnp.int32),
                   pltpu.VMEM((ICR, 128), jnp.int32),
                   pltpu.VMEM((ICR, 128), jnp.int32),
                   pltpu.VMEM((ICR, 128), jnp.int32),
                   pltpu.VMEM_SHARED((ITEM_ROWS, D), jnp.float32),
                   pltpu.SemaphoreType.DMA,
                   pltpu.SemaphoreType.DMA],
)
def _item_scatter(utab, ridx, cidx, zero64, out, rows0, rows1, rawr0, rawr1,
                  rawc0, rawc1, acc, semg, semw):
    c = lax.axis_index("c")
    s = lax.axis_index("s")
    s0 = s * ITEM_STRIPE
    pltpu.sync_copy(zero64.at[pl.ds(0, ITEM_STRIPE)],
                    acc.at[pl.ds(s0, ITEM_STRIPE)])
    plsc.subcore_barrier()

    half0 = c * ITEM_HALF
    sets = ((rows0, rawr0, rawc0), (rows1, rawr1, rawc1))

    @pl.loop(0, PER_SC // ICH // 2)
    def _chunk(i):
        gd = []
        for b, (rows, rawr, rawc) in enumerate(sets):
            r0 = s * (PER_SC // 128) + (2 * i + b) * ICR
            pltpu.sync_copy(ridx.at[pl.ds(r0, ICR)], rawr)
            pltpu.sync_copy(cidx.at[pl.ds(r0, ICR)], rawc)
            for j in range(ICR):
                gd.append(
                    pltpu.async_copy(utab.at[rawr.at[j]],
                                     rows.at[pl.ds(j * 128, 128)], semg))
        for rows, rawr, rawc in sets:
            for j in range(ICR):
                for g in range(8):
                    v = rawc[j, pl.ds(g * 16, 16)]
                    lv = v - half0
                    ok = (lv >= 0) & (lv < ITEM_HALF)
                    rawc[j, pl.ds(g * 16, 16)] = jnp.where(ok, lv, ITEM_TRASH)
        for d in gd:
            d.wait()
        sd = []
        for rows, rawr, rawc in sets:
            for j in range(ICR):
                sd.append(
                    pltpu.async_copy(rows.at[pl.ds(j * 128, 128)],
                                     acc.at[rawc.at[j]], semw, add=True))
        for d in sd:
            d.wait()

    plsc.subcore_barrier()
    pltpu.sync_copy(acc.at[pl.ds(s0, ITEM_STRIPE)],
                    out.at[c, pl.ds(s0, ITEM_STRIPE)])


UCH = 128
UCR = UCH // 128


@functools.partial(
    pl.kernel,
    out_type=jax.ShapeDtypeStruct((NC, USR_ROWS, D), jnp.float32),
    mesh=_MESH,
    compiler_params=_SC_PARAMS,
    scratch_types=[pltpu.VMEM((UCH, D), jnp.float32),
                   pltpu.VMEM((UCH, D), jnp.float32),
                   pltpu.VMEM((UCR, 128), jnp.int32),
                   pltpu.VMEM((UCR, 128), jnp.int32),
                   pltpu.VMEM((UCR, 128), jnp.int32),
                   pltpu.VMEM((UCR, 128), jnp.int32),
                   pltpu.VMEM((UCR, 128), jnp.float32),
                   pltpu.VMEM((UCR, 128), jnp.float32),
                   pltpu.VMEM_SHARED((USR_ROWS, D), jnp.float32),
                   pltpu.SemaphoreType.DMA,
                   pltpu.SemaphoreType.DMA],
)
def _user_scatter(ftab, cidx, ridx, vals, zero64, out, rows0, rows1, rawc0,
                  rawc1, rawr0, rawr1, valv0, valv1, acc, semg, semw):
    c = lax.axis_index("c")
    s = lax.axis_index("s")
    s0 = s * USR_STRIPE
    pltpu.sync_copy(zero64.at[pl.ds(0, USR_STRIPE)],
                    acc.at[pl.ds(s0, USR_STRIPE)])
    plsc.subcore_barrier()

    half0 = c * USR_HALF
    sets = ((rows0, rawc0, rawr0, valv0), (rows1, rawc1, rawr1, valv1))

    @pl.loop(0, PER_SC // UCH // 2)
    def _chunk(i):
        gd = []
        for b, (rows, rawc, rawr, valv) in enumerate(sets):
            r0 = s * (PER_SC // 128) + 2 * i + b
            pltpu.sync_copy(cidx.at[pl.ds(r0, UCR)], rawc)
            pltpu.sync_copy(ridx.at[pl.ds(r0, UCR)], rawr)
            pltpu.sync_copy(vals.at[pl.ds(r0, UCR)], valv)
            gd.append(pltpu.async_copy(ftab.at[rawc.at[0]], rows, semg))
        for rows, rawc, rawr, valv in sets:
            for g in range(8):
                v = rawr[0, pl.ds(g * 16, 16)]
                lv = v - half0
                ok = (lv >= 0) & (lv < USR_HALF)
                rawr[0, pl.ds(g * 16, 16)] = jnp.where(ok, lv, USR_TRASH)
        for d in gd:
            d.wait()
        sd = []
        for rows, rawc, rawr, valv in sets:

            @pl.loop(0, 8)
            def _b(b):
                vals16 = valv[0, pl.ds(b * 16, 16)]

                @pl.loop(0, 16, unroll=4)
                def _r(r):
                    vv = vals16.at[jnp.full((16,), 0, jnp.int32) + r].get(
                        mode="promise_in_bounds")
                    rr = b * 16 + r
                    for g in range(4):
                        rows[rr, pl.ds(g * 16, 16)] = (
                            rows[rr, pl.ds(g * 16, 16)] * vv)

            sd.append(pltpu.async_copy(rows, acc.at[rawr.at[0]], semw,
                                       add=True))
        for d in sd:
            d.wait()

    plsc.subcore_barrier()
    pltpu.sync_copy(acc.at[pl.ds(s0, USR_STRIPE)],
                    out.at[c, pl.ds(s0, USR_STRIPE)])


def _norm(x):
    return jnp.clip(jnp.sqrt(jnp.sum(x * x, axis=-1, keepdims=True)),
                    MIN_NORM, None)


def _lam(p):
    return 2.0 / jnp.clip(1.0 - jnp.sum(p * p, axis=-1, keepdims=True),
                          MIN_NORM, None)


def _mobius_add(x, y):
    x2 = jnp.sum(x * x, axis=-1, keepdims=True)
    y2 = jnp.sum(y * y, axis=-1, keepdims=True)
    xy = jnp.sum(x * y, axis=-1, keepdims=True)
    num = (1.0 + 2.0 * xy + y2) * x + (1.0 - x2) * y
    den = 1.0 + 2.0 * xy + x2 * y2
    return num / jnp.clip(den, MIN_NORM, None)


def _artanh(x):
    xc = jnp.clip(x, -1.0 + 1e-7, 1.0 - 1e-7)
    return 0.5 * jnp.log((1.0 + xc) / (1.0 - xc))


MB = 2048


def _math_body(h_ref, t_ref, ids_ref, w_ref, out_ref):
    h = h_ref[...]
    t = t_ref[...]
    ids = ids_ref[...]
    oneh = (ids - 1 == lax.broadcasted_iota(jnp.int32, (1, N_REL), 1)
            ).astype(jnp.float32)
    r = jnp.dot(oneh, w_ref[...], preferred_element_type=jnp.float32)

    nh = _norm(h)
    p = jnp.tanh(nh) * h / nh
    lam_p = _lam(p)

    nt = _norm(t)
    ht = _mobius_add(p, jnp.tanh(lam_p * nt / 2.0) * t / nt)
    nr = _norm(r)
    hr = _mobius_add(p, jnp.tanh(lam_p * nr / 2.0) * r / nr)

    res = _mobius_add(ht, hr)
    n = _norm(res)
    maxnorm = 1.0 - EPS
    res = jnp.where(n > maxnorm, res * (maxnorm / n), res)

    sub = _mobius_add(-p, res)
    ns = _norm(sub)
    out_ref[...] = (2.0 / lam_p) * _artanh(ns) * sub / ns


def _math_call(h, t, ids, relw):
    grid = EP // MB
    return pl.pallas_call(
        _math_body,
        grid=(grid,),
        in_specs=[pl.BlockSpec((MB, D), lambda i: (i, 0)),
                  pl.BlockSpec((MB, D), lambda i: (i, 0)),
                  pl.BlockSpec((MB, 1), lambda i: (i, 0)),
                  pl.BlockSpec((N_REL, D), lambda i: (0, 0))],
        out_specs=pl.BlockSpec((MB, D), lambda i: (i, 0)),
        out_shape=jax.ShapeDtypeStruct((EP, D), jnp.float32),
    )(h, t, ids, relw)


FB = 2000


def _fusion_body(cf_ref, kg_ref, w1t_ref, w2t_ref, out_ref):
    cf = cf_ref[...]
    kg = kg_ref[...]
    z = (jnp.dot(cf, w1t_ref[...], preferred_element_type=jnp.float32)
         + jnp.dot(kg, w2t_ref[...], preferred_element_type=jnp.float32))
    gi = 1.0 / (1.0 + jnp.exp(-z))
    out_ref[...] = gi * cf + (1.0 - gi) * kg


def _fusion_call(cf, kg, w1t, w2t):
    grid = N_ITEMS // FB
    return pl.pallas_call(
        _fusion_body,
        grid=(grid,),
        in_specs=[pl.BlockSpec((FB, D), lambda i: (i, 0)),
                  pl.BlockSpec((FB, D), lambda i: (i, 0)),
                  pl.BlockSpec((D, D), lambda i: (0, 0)),
                  pl.BlockSpec((D, D), lambda i: (0, 0))],
        out_specs=pl.BlockSpec((FB, D), lambda i: (i, 0)),
        out_shape=jax.ShapeDtypeStruct((N_ITEMS, D), jnp.float32),
    )(cf, kg, w1t, w2t)


def _mean_body(sums_ref, cnt_ref, out_ref):
    cnt = jnp.sum(cnt_ref[...], axis=1, keepdims=True)
    out_ref[...] = sums_ref[...] / jnp.clip(cnt, 1.0, None)


def _mean_call(sums, cnt_parts_t):
    grid = N_ENTITIES // FB
    return pl.pallas_call(
        _mean_body,
        grid=(grid,),
        in_specs=[pl.BlockSpec((FB, D), lambda i: (i, 0)),
                  pl.BlockSpec((FB, NW), lambda i: (i, 0))],
        out_specs=pl.BlockSpec((FB, D), lambda i: (i, 0)),
        out_shape=jax.ShapeDtypeStruct((N_ENTITIES, D), jnp.float32),
    )(sums, cnt_parts_t)


def kernel(entity_emb, user_emb, item_emb_cf, edge_index, edge_type,
           mat_indices, mat_values, relation_weight, W1, W2):
    pad = EP - N_EDGES
    i32 = jnp.int32
    head = edge_index[0]
    tail = edge_index[1]
    mrow = mat_indices[0]
    mcol = mat_indices[1]

    def pad2d(x, fill):
        return jnp.concatenate(
            [x, jnp.full((pad,), fill, x.dtype)]).reshape(EP // 128, 128)

    head_g = pad2d(head, 0)
    tail_g = pad2d(tail, 0)
    head_s = pad2d(head, N_ENTITIES)
    mrow_g = pad2d(mrow, 0)
    mcol_g = pad2d(mcol, 0)
    mrow_s = pad2d(mrow, N_USERS)
    mcol_s = pad2d(mcol, N_ITEMS)
    vals_p = pad2d(mat_values, 0.0)
    etype_p = jnp.concatenate(
        [edge_type, jnp.ones((pad,), i32)]).reshape(EP, 1)

    zero64 = jnp.zeros((ENT_STRIPE, D), jnp.float32)
    zero_cnt = jnp.zeros((CNT_BINS,), jnp.float32)

    fusion = _fusion_call(item_emb_cf, entity_emb[:N_ITEMS], W1.T, W2.T)
    head_rows, tail_rows, cnt_parts = _gather_ht(entity_emb, head_g, tail_g,
                                                 head_s, zero_cnt)
    item_pad = _item_scatter(user_emb, mrow_g, mcol_s, zero64)
    res = _math_call(head_rows, tail_rows, etype_p, relation_weight)
    sums_pad = _ent_scatter(res, head_s, zero64)
    user_pad = _user_scatter(fusion, mcol_g, mrow_s, vals_p, zero64)

    sums = jnp.concatenate([sums_pad[0, :ENT_HALF], sums_pad[1, :ENT_HALF]])
    entity_agg = _mean_call(sums, cnt_parts.T)
    user_agg = jnp.concatenate(
        [user_pad[0, :USR_HALF], user_pad[1, :USR_HALF]])
    item_agg_cf = jnp.concatenate(
        [item_pad[0, :ITEM_HALF], item_pad[1, :ITEM_HALF]])
    return (entity_agg, user_agg, item_agg_cf)


# speedup vs baseline: 1.0181x; 1.0181x over previous
import functools

import jax
import jax.numpy as jnp
from jax import lax
from jax.experimental import pallas as pl
from jax.experimental.pallas import tpu as pltpu
from jax.experimental.pallas import tpu_sc as plsc

MIN_NORM = 1e-15
EPS = 1e-5
N_USERS = 50000
N_ITEMS = 30000
N_ENTITIES = 50000
N_EDGES = 800000
NNZ = 800000
N_REL = 16
D = 64

NC, NS = 2, 16
NW = NC * NS
GCH = 1024
GCR = GCH // 128
GCHUNKS = 25
EP = NW * GCHUNKS * GCH
PER_SC = EP // NS

ENT_HALF = 25000
ENT_ROWS = 25088
ENT_STRIPE = ENT_ROWS // NS
ENT_TRASH = 25040
ITEM_HALF = 15000
ITEM_ROWS = 15104
ITEM_STRIPE = ITEM_ROWS // NS
ITEM_TRASH = 15040
USR_HALF = 25000
USR_ROWS = 25088
USR_STRIPE = 1568
USR_TRASH = 25040

CNT_BINS = 50176

_MESH = plsc.VectorSubcoreMesh(core_axis_name="c", subcore_axis_name="s")
_SC_PARAMS = pltpu.CompilerParams(use_tc_tiling_on_sc=False,
                                  needs_layout_passes=False)


def _wid():
    return lax.axis_index("s") * NC + lax.axis_index("c")


GW = 512
GWR = GW // 128


@functools.partial(
    pl.kernel,
    out_type=[jax.ShapeDtypeStruct((EP, D), jnp.float32),
              jax.ShapeDtypeStruct((EP, D), jnp.float32),
              jax.ShapeDtypeStruct((NW, CNT_BINS), jnp.float32)],
    mesh=_MESH,
    compiler_params=_SC_PARAMS,
    scratch_types=[pltpu.VMEM((GWR, 128), jnp.int32),
                   pltpu.VMEM((GWR, 128), jnp.int32),
                   pltpu.VMEM((2 * GWR, 128), jnp.int32),
                   pltpu.VMEM((CNT_BINS,), jnp.float32),
                   pltpu.VMEM((GW, D), jnp.float32),
                   pltpu.VMEM((GW, D), jnp.float32),
                   pltpu.SemaphoreType.DMA,
                   pltpu.SemaphoreType.DMA],
)
def _gather_ht(tab, hidx, tidx, hidx_s, zero_cnt, out_h, out_t, out_cnt,
               ib0, ib1, ibs, hist, b0, b1, semg, semw):
    w = _wid()
    base_r = w * (EP // NW // 128)
    pltpu.sync_copy(zero_cnt, hist)
    ones = jnp.ones((16,), jnp.float32)

    for idx_ref, out_ref, do_hist in ((hidx, out_h, True),
                                      (tidx, out_t, False)):

        @pl.loop(0, 25)
        def _i(i):
            r0 = base_r + i * 2 * GWR
            pltpu.sync_copy(idx_ref.at[pl.ds(r0, GWR)], ib0)
            pltpu.sync_copy(idx_ref.at[pl.ds(r0 + GWR, GWR)], ib1)
            if do_hist:
                pltpu.sync_copy(hidx_s.at[pl.ds(r0, 2 * GWR)], ibs)

            @pl.when(i > 0)
            def _drain_writes():
                pltpu.make_async_copy(b0, out_ref.at[pl.ds(0, GW)],
                                      semw).wait()
                pltpu.make_async_copy(b1, out_ref.at[pl.ds(0, GW)],
                                      semw).wait()

            descs = []
            for ib, buf in ((ib0, b0), (ib1, b1)):
                for j in range(GWR):
                    descs.append(
                        pltpu.async_copy(tab.at[ib.at[j]],
                                         buf.at[pl.ds(j * 128, 128)], semg))
            if do_hist:
                for j in range(2 * GWR):

                    @pl.loop(0, 8, unroll=2)
                    def _g(g):
                        v = ibs[j, pl.ds(g * 16, 16)]
                        plsc.addupdate_scatter(hist, [v], ones)

            for d in descs:
                d.wait()
            pltpu.async_copy(b0, out_ref.at[pl.ds(r0 * 128, GW)], semw)
            pltpu.async_copy(b1, out_ref.at[pl.ds((r0 + GWR) * 128, GW)],
                             semw)

        pltpu.make_async_copy(b0, out_ref.at[pl.ds(0, GW)], semw).wait()
        pltpu.make_async_copy(b1, out_ref.at[pl.ds(0, GW)], semw).wait()

    pltpu.sync_copy(hist, out_cnt.at[w])


ECH = 256
ECR = ECH // 128


@functools.partial(
    pl.kernel,
    out_type=jax.ShapeDtypeStruct((NC, ENT_ROWS, D), jnp.float32),
    mesh=_MESH,
    compiler_params=_SC_PARAMS,
    scratch_types=[pltpu.VMEM((ECH, D), jnp.float32),
                   pltpu.VMEM((ECR, 128), jnp.int32),
                   pltpu.VMEM((ECR, 128), jnp.int32),
                   pltpu.VMEM_SHARED((ENT_ROWS, D), jnp.float32),
                   pltpu.SemaphoreType.DMA],
)
def _ent_scatter(res, hidx, zero64, sums_out, rows, rawi, locv, sums_acc,
                 semw):
    c = lax.axis_index("c")
    s = lax.axis_index("s")
    s0 = s * ENT_STRIPE
    pltpu.sync_copy(zero64, sums_acc.at[pl.ds(s0, ENT_STRIPE)])
    plsc.subcore_barrier()

    half0 = c * ENT_HALF

    @pl.loop(0, PER_SC // ECH)
    def _chunk(k):
        base = s * PER_SC + k * ECH
        r0 = base // 128
        pltpu.sync_copy(res.at[pl.ds(base, ECH)], rows)
        pltpu.sync_copy(hidx.at[pl.ds(r0, ECR)], rawi)
        for j in range(ECR):
            for g in range(8):
                v = rawi[j, pl.ds(g * 16, 16)]
                lv = v - half0
                ok = (lv >= 0) & (lv < ENT_HALF)
                locv[j, pl.ds(g * 16, 16)] = jnp.where(ok, lv, ENT_TRASH)
        sd = [
            pltpu.async_copy(rows.at[pl.ds(j * 128, 128)],
                             sums_acc.at[locv.at[j]], semw, add=True)
            for j in range(ECR)
        ]
        for d in sd:
            d.wait()

    plsc.subcore_barrier()
    pltpu.sync_copy(sums_acc.at[pl.ds(s0, ENT_STRIPE)],
                    sums_out.at[c, pl.ds(s0, ENT_STRIPE)])


ICH = 512
ICR = ICH // 128


@functools.partial(
    pl.kernel,
    out_type=jax.ShapeDtypeStruct((NC, ITEM_ROWS, D), jnp.float32),
    mesh=_MESH,
    compiler_params=_SC_PARAMS,
    scratch_types=[pltpu.VMEM((ICH, D), jnp.float32),
                   pltpu.VMEM((ICH, D), jnp.float32),
                   pltpu.VMEM((ICR, 128), jnp.int32),
                   pltpu.VMEM((ICR, 128), jnp.int32),
                   pltpu.VMEM((ICR, 128), jnp.int32),
                   pltpu.VMEM((ICR, 128), jnp.int32),
                   pltpu.VMEM_SHARED((ITEM_ROWS, D), jnp.float32),
                   pltpu.SemaphoreType.DMA,
                   pltpu.SemaphoreType.DMA],
)
def _item_scatter(utab, ridx, cidx, zero64, out, rows0, rows1, rawr0, rawr1,
                  rawc0, rawc1, acc, semg, semw):
    c = lax.axis_index("c")
    s = lax.axis_index("s")
    s0 = s * ITEM_STRIPE
    pltpu.sync_copy(zero64.at[pl.ds(0, ITEM_STRIPE)],
                    acc.at[pl.ds(s0, ITEM_STRIPE)])
    plsc.subcore_barrier()

    half0 = c * ITEM_HALF
    sets = ((rows0, rawr0, rawc0), (rows1, rawr1, rawc1))

    @pl.loop(0, PER_SC // ICH // 2)
    def _chunk(i):
        gd = []
        for b, (rows, rawr, rawc) in enumerate(sets):
            r0 = s * (PER_SC // 128) + (2 * i + b) * ICR
            pltpu.sync_copy(ridx.at[pl.ds(r0, ICR)], rawr)
            pltpu.sync_copy(cidx.at[pl.ds(r0, ICR)], rawc)
            for j in range(ICR):
                gd.append(
                    pltpu.async_copy(utab.at[rawr.at[j]],
                                     rows.at[pl.ds(j * 128, 128)], semg))
        for rows, rawr, rawc in sets:
            for j in range(ICR):
                for g in range(8):
                    v = rawc[j, pl.ds(g * 16, 16)]
                    lv = v - half0
                    ok = (lv >= 0) & (lv < ITEM_HALF)
                    rawc[j, pl.ds(g * 16, 16)] = jnp.where(ok, lv, ITEM_TRASH)
        for d in gd:
            d.wait()
        sd = []
        for rows, rawr, rawc in sets:
            for j in range(ICR):
                sd.append(
                    pltpu.async_copy(rows.at[pl.ds(j * 128, 128)],
                                     acc.at[rawc.at[j]], semw, add=True))
        for d in sd:
            d.wait()

    plsc.subcore_barrier()
    pltpu.sync_copy(acc.at[pl.ds(s0, ITEM_STRIPE)],
                    out.at[c, pl.ds(s0, ITEM_STRIPE)])


UCH = 128
UCR = UCH // 128


@functools.partial(
    pl.kernel,
    out_type=jax.ShapeDtypeStruct((NC, USR_ROWS, D), jnp.float32),
    mesh=_MESH,
    compiler_params=_SC_PARAMS,
    scratch_types=[pltpu.VMEM((UCH, D), jnp.float32),
                   pltpu.VMEM((UCH, D), jnp.float32),
                   pltpu.VMEM((UCR, 128), jnp.int32),
                   pltpu.VMEM((UCR, 128), jnp.int32),
                   pltpu.VMEM((UCR, 128), jnp.int32),
                   pltpu.VMEM((UCR, 128), jnp.int32),
                   pltpu.VMEM((UCR, 128), jnp.float32),
                   pltpu.VMEM((UCR, 128), jnp.float32),
                   pltpu.VMEM_SHARED((USR_ROWS, D), jnp.float32),
                   pltpu.SemaphoreType.DMA,
                   pltpu.SemaphoreType.DMA],
)
def _user_scatter(ftab, cidx, ridx, vals, zero64, out, rows0, rows1, rawc0,
                  rawc1, rawr0, rawr1, valv0, valv1, acc, semg, semw):
    c = lax.axis_index("c")
    s = lax.axis_index("s")
    s0 = s * USR_STRIPE
    pltpu.sync_copy(zero64.at[pl.ds(0, USR_STRIPE)],
                    acc.at[pl.ds(s0, USR_STRIPE)])
    plsc.subcore_barrier()

    half0 = c * USR_HALF
    sets = ((rows0, rawc0, rawr0, valv0), (rows1, rawc1, rawr1, valv1))

    @pl.loop(0, PER_SC // UCH // 2)
    def _chunk(i):
        gd = []
        for b, (rows, rawc, rawr, valv) in enumerate(sets):
            r0 = s * (PER_SC // 128) + 2 * i + b
            pltpu.sync_copy(cidx.at[pl.ds(r0, UCR)], rawc)
            pltpu.sync_copy(ridx.at[pl.ds(r0, UCR)], rawr)
            pltpu.sync_copy(vals.at[pl.ds(r0, UCR)], valv)
            gd.append(pltpu.async_copy(ftab.at[rawc.at[0]], rows, semg))
        for rows, rawc, rawr, valv in sets:
            for g in range(8):
                v = rawr[0, pl.ds(g * 16, 16)]
                lv = v - half0
                ok = (lv >= 0) & (lv < USR_HALF)
                rawr[0, pl.ds(g * 16, 16)] = jnp.where(ok, lv, USR_TRASH)
        for d in gd:
            d.wait()
        sd = []
        for rows, rawc, rawr, valv in sets:

            @pl.loop(0, 8)
            def _b(b):
                vals16 = valv[0, pl.ds(b * 16, 16)]

                @pl.loop(0, 16, unroll=4)
                def _r(r):
                    vv = vals16.at[jnp.full((16,), 0, jnp.int32) + r].get(
                        mode="promise_in_bounds")
                    rr = b * 16 + r
                    for g in range(4):
                        rows[rr, pl.ds(g * 16, 16)] = (
                            rows[rr, pl.ds(g * 16, 16)] * vv)

            sd.append(pltpu.async_copy(rows, acc.at[rawr.at[0]], semw,
                                       add=True))
        for d in sd:
            d.wait()

    plsc.subcore_barrier()
    pltpu.sync_copy(acc.at[pl.ds(s0, USR_STRIPE)],
                    out.at[c, pl.ds(s0, USR_STRIPE)])


def _norm(x):
    return jnp.clip(jnp.sqrt(jnp.sum(x * x, axis=-1, keepdims=True)),
                    MIN_NORM, None)


def _lam(p):
    return 2.0 / jnp.clip(1.0 - jnp.sum(p * p, axis=-1, keepdims=True),
                          MIN_NORM, None)


def _mobius_add(x, y):
    x2 = jnp.sum(x * x, axis=-1, keepdims=True)
    y2 = jnp.sum(y * y, axis=-1, keepdims=True)
    xy = jnp.sum(x * y, axis=-1, keepdims=True)
    num = (1.0 + 2.0 * xy + y2) * x + (1.0 - x2) * y
    den = 1.0 + 2.0 * xy + x2 * y2
    return num / jnp.clip(den, MIN_NORM, None)


def _artanh(x):
    xc = jnp.clip(x, -1.0 + 1e-7, 1.0 - 1e-7)
    return 0.5 * jnp.log((1.0 + xc) / (1.0 - xc))


MB = 2048


def _math_body(h_ref, t_ref, ids_ref, w_ref, out_ref):
    h = h_ref[...]
    t = t_ref[...]
    ids = ids_ref[...]
    oneh = (ids - 1 == lax.broadcasted_iota(jnp.int32, (1, N_REL), 1)
            ).astype(jnp.float32)
    r = jnp.dot(oneh, w_ref[...], preferred_element_type=jnp.float32)

    nh = _norm(h)
    p = jnp.tanh(nh) * h / nh
    lam_p = _lam(p)

    nt = _norm(t)
    ht = _mobius_add(p, jnp.tanh(lam_p * nt / 2.0) * t / nt)
    nr = _norm(r)
    hr = _mobius_add(p, jnp.tanh(lam_p * nr / 2.0) * r / nr)

    res = _mobius_add(ht, hr)
    n = _norm(res)
    maxnorm = 1.0 - EPS
    res = jnp.where(n > maxnorm, res * (maxnorm / n), res)

    sub = _mobius_add(-p, res)
    ns = _norm(sub)
    out_ref[...] = (2.0 / lam_p) * _artanh(ns) * sub / ns


def _math_call(h, t, ids, relw):
    grid = EP // MB
    return pl.pallas_call(
        _math_body,
        grid=(grid,),
        in_specs=[pl.BlockSpec((MB, D), lambda i: (i, 0)),
                  pl.BlockSpec((MB, D), lambda i: (i, 0)),
                  pl.BlockSpec((MB, 1), lambda i: (i, 0)),
                  pl.BlockSpec((N_REL, D), lambda i: (0, 0))],
        out_specs=pl.BlockSpec((MB, D), lambda i: (i, 0)),
        out_shape=jax.ShapeDtypeStruct((EP, D), jnp.float32),
    )(h, t, ids, relw)


FB = 2000


def _fusion_body(cf_ref, kg_ref, w1t_ref, w2t_ref, out_ref):
    cf = cf_ref[...]
    kg = kg_ref[...]
    z = (jnp.dot(cf, w1t_ref[...], preferred_element_type=jnp.float32)
         + jnp.dot(kg, w2t_ref[...], preferred_element_type=jnp.float32))
    gi = 1.0 / (1.0 + jnp.exp(-z))
    out_ref[...] = gi * cf + (1.0 - gi) * kg


def _fusion_call(cf, kg, w1t, w2t):
    grid = N_ITEMS // FB
    return pl.pallas_call(
        _fusion_body,
        grid=(grid,),
        in_specs=[pl.BlockSpec((FB, D), lambda i: (i, 0)),
                  pl.BlockSpec((FB, D), lambda i: (i, 0)),
                  pl.BlockSpec((D, D), lambda i: (0, 0)),
                  pl.BlockSpec((D, D), lambda i: (0, 0))],
        out_specs=pl.BlockSpec((FB, D), lambda i: (i, 0)),
        out_shape=jax.ShapeDtypeStruct((N_ITEMS, D), jnp.float32),
    )(cf, kg, w1t, w2t)


def _mean_body(sums_ref, cnt_ref, out_ref):
    cnt = jnp.sum(cnt_ref[...], axis=1, keepdims=True)
    out_ref[...] = sums_ref[...] / jnp.clip(cnt, 1.0, None)


def _mean_call(sums, cnt_parts_t):
    grid = N_ENTITIES // FB
    return pl.pallas_call(
        _mean_body,
        grid=(grid,),
        in_specs=[pl.BlockSpec((FB, D), lambda i: (i, 0)),
                  pl.BlockSpec((FB, NW), lambda i: (i, 0))],
        out_specs=pl.BlockSpec((FB, D), lambda i: (i, 0)),
        out_shape=jax.ShapeDtypeStruct((N_ENTITIES, D), jnp.float32),
    )(sums, cnt_parts_t)


def kernel(entity_emb, user_emb, item_emb_cf, edge_index, edge_type,
           mat_indices, mat_values, relation_weight, W1, W2):
    pad = EP - N_EDGES
    i32 = jnp.int32
    head = edge_index[0]
    tail = edge_index[1]
    mrow = mat_indices[0]
    mcol = mat_indices[1]

    def pad2d(x, fill):
        return jnp.concatenate(
            [x, jnp.full((pad,), fill, x.dtype)]).reshape(EP // 128, 128)

    head_g = pad2d(head, 0)
    tail_g = pad2d(tail, 0)
    head_s = pad2d(head, N_ENTITIES)
    mrow_g = pad2d(mrow, 0)
    mcol_g = pad2d(mcol, 0)
    mrow_s = pad2d(mrow, N_USERS)
    mcol_s = pad2d(mcol, N_ITEMS)
    vals_p = pad2d(mat_values, 0.0)
    etype_p = jnp.concatenate(
        [edge_type, jnp.ones((pad,), i32)]).reshape(EP, 1)

    zero64 = jnp.zeros((ENT_STRIPE, D), jnp.float32)
    zero_cnt = jnp.zeros((CNT_BINS,), jnp.float32)

    fusion = _fusion_call(item_emb_cf, entity_emb[:N_ITEMS], W1.T, W2.T)
    head_rows, tail_rows, cnt_parts = _gather_ht(entity_emb, head_g, tail_g,
                                                 head_s, zero_cnt)
    res = _math_call(head_rows, tail_rows, etype_p, relation_weight)
    sums_pad = _ent_scatter(res, head_s, zero64)
    item_pad = _item_scatter(user_emb, mrow_g, mcol_s, zero64)
    user_pad = _user_scatter(fusion, mcol_g, mrow_s, vals_p, zero64)

    sums = jnp.concatenate([sums_pad[0, :ENT_HALF], sums_pad[1, :ENT_HALF]])
    entity_agg = _mean_call(sums, cnt_parts.T)
    user_agg = jnp.concatenate(
        [user_pad[0, :USR_HALF], user_pad[1, :USR_HALF]])
    item_agg_cf = jnp.concatenate(
        [item_pad[0, :ITEM_HALF], item_pad[1, :ITEM_HALF]])
    return (entity_agg, user_agg, item_agg_cf)
